# baseline (speedup 1.0000x reference)
import functools

import jax
import jax.numpy as jnp
from jax import lax
from jax.experimental import pallas as pl
from jax.experimental.pallas import tpu as pltpu
from jax.experimental.pallas import tpu_sc as plsc

_NC = 2
_NS = 16
_NW = _NC * _NS

_BATCH = 1024
_SEQ = 200
_NTOK = 10
_DIM = 64
_OUT_S = _NTOK + _SEQ
_BPW = _BATCH // _NW

_C0 = 104
_C1 = _SEQ - _C0


def _make_kernel():
    mesh = plsc.VectorSubcoreMesh(core_axis_name="c", subcore_axis_name="s")

    @functools.partial(
        pl.kernel,
        out_type=jax.ShapeDtypeStruct((_BATCH, _OUT_S, _DIM), jnp.float32),
        mesh=mesh,
        scratch_types=[
            pltpu.VMEM((_SEQ,), jnp.int32),
            pltpu.VMEM((_OUT_S, _DIM), jnp.float32),
            pltpu.SemaphoreType.DMA,
        ],
        compiler_params=pltpu.CompilerParams(use_tc_tiling_on_sc=False),
    )
    def prompt_embed(tokens_hbm, table_hbm, learned_hbm, out_hbm,
                     idx_v, buf_v, sem):
        wid = lax.axis_index("s") * _NC + lax.axis_index("c")
        b0 = wid * _BPW
        pltpu.sync_copy(learned_hbm, buf_v.at[pl.ds(0, _NTOK)])

        @pl.loop(0, _BPW)
        def _body(i):
            b = b0 + i
            pltpu.sync_copy(tokens_hbm.at[pl.ds(b * _SEQ, _SEQ)], idx_v)
            g0 = pltpu.async_copy(
                table_hbm.at[idx_v.at[pl.ds(0, _C0)]],
                buf_v.at[pl.ds(_NTOK, _C0)], sem)
            g1 = pltpu.async_copy(
                table_hbm.at[idx_v.at[pl.ds(_C0, _C1)]],
                buf_v.at[pl.ds(_NTOK + _C0, _C1)], sem)
            g0.wait()
            g1.wait()
            pltpu.sync_copy(buf_v, out_hbm.at[b])

    return prompt_embed


_PROMPT_EMBED = _make_kernel()


def kernel(tokens, wte_weight, learned_embedding):
    return _PROMPT_EMBED(
        tokens.reshape(-1).astype(jnp.int32), wte_weight, learned_embedding)


# speedup vs baseline: 1.0382x; 1.0382x over previous
import functools

import jax
import jax.numpy as jnp
from jax import lax
from jax.experimental import pallas as pl
from jax.experimental.pallas import tpu as pltpu
from jax.experimental.pallas import tpu_sc as plsc

_NC = 2
_NS = 16
_NW = _NC * _NS

_BATCH = 1024
_SEQ = 200
_NTOK = 10
_DIM = 64
_OUT_S = _NTOK + _SEQ
_BPW = _BATCH // _NW

_C0 = 104
_C1 = _SEQ - _C0

_K = 4
_NSLOT = 2 * _K
_NGRP = _BPW // _K


def _make_kernel():
    mesh = plsc.VectorSubcoreMesh(core_axis_name="c", subcore_axis_name="s")

    @functools.partial(
        pl.kernel,
        out_type=jax.ShapeDtypeStruct((_BATCH, _OUT_S, _DIM), jnp.float32),
        mesh=mesh,
        scratch_types=[
            pltpu.VMEM((_BPW * _SEQ,), jnp.int32),
            pltpu.VMEM((_NSLOT, _OUT_S, _DIM), jnp.float32),
            pltpu.SemaphoreType.DMA,
            pltpu.SemaphoreType.DMA,
            pltpu.SemaphoreType.DMA,
            pltpu.SemaphoreType.DMA,
        ],
        compiler_params=pltpu.CompilerParams(use_tc_tiling_on_sc=False),
    )
    def prompt_embed(tokens_hbm, table_hbm, learned_hbm, out_hbm,
                     idx_v, bufs, gsemA, gsemB, ssemA, ssemB):
        wid = lax.axis_index("s") * _NC + lax.axis_index("c")
        b0 = wid * _BPW

        pltpu.sync_copy(tokens_hbm.at[pl.ds(b0 * _SEQ, _BPW * _SEQ)], idx_v)
        for s in range(_NSLOT):
            pltpu.sync_copy(learned_hbm, bufs.at[s, pl.ds(0, _NTOK)])

        def issue_gathers(g, slot_base, gsem):
            for s in range(_K):
                l = g * _K + s
                pltpu.async_copy(
                    table_hbm.at[idx_v.at[pl.ds(l * _SEQ, _C0)]],
                    bufs.at[slot_base + s, pl.ds(_NTOK, _C0)], gsem)
                pltpu.async_copy(
                    table_hbm.at[idx_v.at[pl.ds(l * _SEQ + _C0, _C1)]],
                    bufs.at[slot_base + s, pl.ds(_NTOK + _C0, _C1)], gsem)

        def drain_issue_stores(g, slot_base, gsem, ssem):
            for s in range(_K):
                l = g * _K + s
                pltpu.make_async_copy(
                    table_hbm.at[idx_v.at[pl.ds(0, _C0)]],
                    bufs.at[slot_base + s, pl.ds(_NTOK, _C0)], gsem).wait()
                pltpu.make_async_copy(
                    table_hbm.at[idx_v.at[pl.ds(0, _C1)]],
                    bufs.at[slot_base + s, pl.ds(_NTOK, _C1)], gsem).wait()
                pltpu.async_copy(bufs.at[slot_base + s], out_hbm.at[b0 + l],
                                 ssem)

        def drain_stores(ssem):
            for s in range(_K):
                pltpu.make_async_copy(bufs.at[s], out_hbm.at[b0], ssem).wait()

        @pl.loop(0, _NGRP, step=2)
        def _body(g):
            @pl.when(g > 0)
            def _():
                drain_stores(ssemA)
            issue_gathers(g, 0, gsemA)
            @pl.when(g > 0)
            def _():
                drain_stores(ssemB)
            drain_issue_stores(g, 0, gsemA, ssemA)
            issue_gathers(g + 1, _K, gsemB)
            drain_issue_stores(g + 1, _K, gsemB, ssemB)

        drain_stores(ssemA)
        drain_stores(ssemB)

    return prompt_embed


_PROMPT_EMBED = _make_kernel()


def kernel(tokens, wte_weight, learned_embedding):
    return _PROMPT_EMBED(
        tokens.reshape(-1).astype(jnp.int32), wte_weight, learned_embedding)
